# baseline (speedup 1.0000x reference)
import functools

import jax
import jax.numpy as jnp
from jax.experimental import pallas as pl
from jax.experimental.pallas import tpu as pltpu

_ARMS = 8


def _mm_body(x_ref, w_ref, b_ref, o_ref):
    o_ref[...] = (
        jnp.dot(x_ref[...], w_ref[...], preferred_element_type=jnp.float32)
        + b_ref[...]
    )


def _project(x, W, b, bn=80, bo=512):
    N, D = x.shape
    O = W.shape[0]
    if D < 128:
        pad = 128 - D
        x = jnp.pad(x, ((0, 0), (0, pad)))
        W = jnp.pad(W, ((0, 0), (0, pad)))
        D = 128
    Wt = W.T
    grid = (N // bn, O // bo)
    return pl.pallas_call(
        _mm_body,
        grid=grid,
        in_specs=[
            pl.BlockSpec((bn, D), lambda i, j: (i, 0)),
            pl.BlockSpec((D, bo), lambda i, j: (0, j)),
            pl.BlockSpec((1, bo), lambda i, j: (0, j)),
        ],
        out_specs=pl.BlockSpec((bn, bo), lambda i, j: (i, j)),
        out_shape=jax.ShapeDtypeStruct((N, O), jnp.float32),
    )(x, Wt, b.reshape(1, O))


def _alpha_body(sd_ref, ea_ref, we_ref, att_ref, *refs):
    xs = refs[: 2 * _ARMS]
    o_ref = refs[2 * _ARMS]
    we = we_ref[...]
    att2 = att_ref[...]
    xl_rows = jnp.concatenate([xs[j][0] for j in range(_ARMS)], axis=0)
    xr_rows = jnp.concatenate([xs[_ARMS + j][0] for j in range(_ARMS)], axis=0)
    ea = ea_ref[0, 0, :].reshape(_ARMS, 1)
    m = xl_rows + xr_rows + ea * we
    m = jnp.where(m > 0, m, 0.2 * m)
    o_ref[0] = jnp.dot(m, att2, preferred_element_type=jnp.float32)


def _edge_alpha(xl, xr, sd, ea_s, we_flat, att_flat):
    Ef = sd.shape[1]
    hc = we_flat.shape[0]
    G = Ef // _ARMS
    ea3 = ea_s.reshape(G, 1, _ARMS)
    n = xl.shape[0]
    xl3 = xl.reshape(n, 1, hc)
    xr3 = xr.reshape(n, 1, hc)

    in_specs = [
        pl.BlockSpec((1, 1, _ARMS), lambda i, sd_r: (i, 0, 0)),
        pl.BlockSpec((1, hc), lambda i, sd_r: (0, 0)),
        pl.BlockSpec((hc, 2), lambda i, sd_r: (0, 0)),
    ]
    for j in range(_ARMS):
        in_specs.append(
            pl.BlockSpec((1, 1, hc), lambda i, sd_r, j=j: (sd_r[0, i * _ARMS + j], 0, 0))
        )
    for j in range(_ARMS):
        in_specs.append(
            pl.BlockSpec((1, 1, hc), lambda i, sd_r, j=j: (sd_r[1, i * _ARMS + j], 0, 0))
        )

    c = hc // 2
    head = (jnp.arange(hc) // c).astype(jnp.int32)
    att2 = jnp.where(
        jnp.stack([head == 0, head == 1], axis=1),
        att_flat[:, None], 0.0).astype(jnp.float32)

    grid_spec = pltpu.PrefetchScalarGridSpec(
        num_scalar_prefetch=1,
        grid=(G,),
        in_specs=in_specs,
        out_specs=pl.BlockSpec((1, _ARMS, 2), lambda i, sd_r: (i, 0, 0)),
    )
    out = pl.pallas_call(
        _alpha_body,
        grid_spec=grid_spec,
        out_shape=jax.ShapeDtypeStruct((G, _ARMS, 2), jnp.float32),
    )(sd, ea3, we_flat.reshape(1, hc), att2, *([xl3] * _ARMS), *([xr3] * _ARMS))
    return out.reshape(Ef, 2)


def _softmax_body(d_ref, al_ref, o_ref):
    d = d_ref[...]
    al = al_ref[...]
    n = d.shape[1]
    neg = jnp.float32(-3e38)

    def fwd(vals, dd, op, fill):
        v = vals
        s = 1
        while s < n:
            pv = jnp.concatenate(
                [jnp.full((2, s), fill, jnp.float32), v[:, :-s]], axis=1)
            pd = jnp.concatenate(
                [jnp.full((2, s), -1, jnp.int32), dd[:, :-s]], axis=1)
            v = jnp.where(dd == pd, op(v, pv), v)
            s *= 2
        return v

    def bwd(vals, dd, op, fill):
        v = vals
        s = 1
        while s < n:
            nv = jnp.concatenate(
                [v[:, s:], jnp.full((2, s), fill, jnp.float32)], axis=1)
            nd = jnp.concatenate(
                [dd[:, s:], jnp.full((2, s), -1, jnp.int32)], axis=1)
            v = jnp.where(dd == nd, op(v, nv), v)
            s *= 2
        return v

    amax = bwd(fwd(al, d, jnp.maximum, neg), d, jnp.maximum, neg)
    ex = jnp.exp(al - amax)
    den = bwd(fwd(ex, d, jnp.add, 0.0), d, jnp.maximum, neg)
    o_ref[...] = ex / (den + 1e-16)


def _seg_softmax(alpha, dst_s):
    Ef = alpha.shape[0]
    al_t = alpha.T
    d_t = jnp.broadcast_to(dst_s[None, :], (2, Ef)).astype(jnp.int32)
    out = pl.pallas_call(
        _softmax_body,
        out_shape=jax.ShapeDtypeStruct((2, Ef), jnp.float32),
    )(d_t, al_t)
    return out.T


def _gat(x, p, sd, ea_s, num_nodes, H, C):
    xl = _project(x, p["Wl"], p["bl"])
    xr = _project(x, p["Wr"], p["br"])
    we_flat = p["We"].reshape(-1)
    att_flat = p["att"].reshape(-1)

    alpha = _edge_alpha(xl, xr, sd, ea_s, we_flat, att_flat)
    dst_s = sd[1]
    amax = jax.ops.segment_max(alpha, dst_s, num_segments=num_nodes)
    amax = jnp.where(jnp.isfinite(amax), amax, 0.0)
    ex = jnp.exp(alpha - amax[dst_s])
    den = jax.ops.segment_sum(ex, dst_s, num_segments=num_nodes)
    a = ex / (den[dst_s] + 1e-16)

    w = jnp.repeat(a, C, axis=1)
    out = jax.ops.segment_sum(xl[sd[0]] * w, sd[1], num_segments=num_nodes)
    out = out.reshape(num_nodes, H, C).mean(axis=1) + p["bias"]
    return out


def _bn(x, g, b):
    m = x.mean(axis=0)
    v = x.var(axis=0)
    return g * (x - m) / jnp.sqrt(v + 1e-5) + b


def kernel(h, edge_index, edge_weight, params):
    num_nodes = h.shape[0]
    H = params["conv1"]["att"].shape[0]
    C = params["conv1"]["att"].shape[1]
    src = edge_index[0]
    dst = edge_index[1]
    loop = jnp.arange(num_nodes, dtype=src.dtype)
    src_f = jnp.concatenate([src, loop])
    dst_f = jnp.concatenate([dst, loop])
    ea_mean = jnp.mean(edge_weight, axis=0, keepdims=True)
    ea_f = jnp.concatenate(
        [edge_weight.reshape(-1), jnp.broadcast_to(ea_mean.reshape(1), (num_nodes,))]
    )

    perm = jnp.argsort(dst_f)
    src_s = src_f[perm]
    dst_s = dst_f[perm]
    ea_s = ea_f[perm]
    sd = jnp.stack([src_s, dst_s]).astype(jnp.int32)

    x = _bn(h, params["bn0_g"], params["bn0_b"])
    x = jax.nn.relu(
        _bn(_gat(x, params["conv1"], sd, ea_s, num_nodes, H, C),
            params["bn1_g"], params["bn1_b"]))
    x = jax.nn.relu(
        _bn(_gat(x, params["conv2"], sd, ea_s, num_nodes, H, C),
            params["bn2_g"], params["bn2_b"]))
    mu = _gat(x, params["mu"], sd, ea_s, num_nodes, H, C)
    log_std = _gat(x, params["log_std"], sd, ea_s, num_nodes, H, C)
    return (mu, log_std)


# speedup vs baseline: 1.1127x; 1.1127x over previous
import functools

import jax
import jax.numpy as jnp
from jax.experimental import pallas as pl
from jax.experimental.pallas import tpu as pltpu

_ARMS = 16


def _mm_body(x_ref, w_ref, b_ref, o_ref):
    o_ref[...] = (
        jnp.dot(x_ref[...], w_ref[...], preferred_element_type=jnp.float32)
        + b_ref[...]
    )


def _project(x, W, b, bn=80, bo=512):
    N, D = x.shape
    O = W.shape[0]
    if D < 128:
        pad = 128 - D
        x = jnp.pad(x, ((0, 0), (0, pad)))
        W = jnp.pad(W, ((0, 0), (0, pad)))
        D = 128
    Wt = W.T
    grid = (N // bn, O // bo)
    return pl.pallas_call(
        _mm_body,
        grid=grid,
        in_specs=[
            pl.BlockSpec((bn, D), lambda i, j: (i, 0)),
            pl.BlockSpec((D, bo), lambda i, j: (0, j)),
            pl.BlockSpec((1, bo), lambda i, j: (0, j)),
        ],
        out_specs=pl.BlockSpec((bn, bo), lambda i, j: (i, j)),
        out_shape=jax.ShapeDtypeStruct((N, O), jnp.float32),
    )(x, Wt, b.reshape(1, O))


def _alpha_body(sd_ref, ea_ref, we_ref, att_ref, *refs):
    xs = refs[: 2 * _ARMS]
    o_ref = refs[2 * _ARMS]
    we = we_ref[...]
    att2 = att_ref[...]
    xl_rows = jnp.concatenate([xs[j][0] for j in range(_ARMS)], axis=0)
    xr_rows = jnp.concatenate([xs[_ARMS + j][0] for j in range(_ARMS)], axis=0)
    ea = ea_ref[0, 0, :].reshape(_ARMS, 1)
    m = xl_rows + xr_rows + ea * we
    m = jnp.where(m > 0, m, 0.2 * m)
    o_ref[0] = jnp.dot(m, att2, preferred_element_type=jnp.float32)


def _edge_alpha(xl, xr, sd, ea_s, we_flat, att_flat):
    Ef = sd.shape[1]
    hc = we_flat.shape[0]
    G = Ef // _ARMS
    ea3 = ea_s.reshape(G, 1, _ARMS)
    n = xl.shape[0]
    xl3 = xl.reshape(n, 1, hc)
    xr3 = xr.reshape(n, 1, hc)

    in_specs = [
        pl.BlockSpec((1, 1, _ARMS), lambda i, sd_r: (i, 0, 0)),
        pl.BlockSpec((1, hc), lambda i, sd_r: (0, 0)),
        pl.BlockSpec((hc, 2), lambda i, sd_r: (0, 0)),
    ]
    for j in range(_ARMS):
        in_specs.append(
            pl.BlockSpec((1, 1, hc), lambda i, sd_r, j=j: (sd_r[0, i * _ARMS + j], 0, 0))
        )
    for j in range(_ARMS):
        in_specs.append(
            pl.BlockSpec((1, 1, hc), lambda i, sd_r, j=j: (sd_r[1, i * _ARMS + j], 0, 0))
        )

    c = hc // 2
    head = (jnp.arange(hc) // c).astype(jnp.int32)
    att2 = jnp.where(
        jnp.stack([head == 0, head == 1], axis=1),
        att_flat[:, None], 0.0).astype(jnp.float32)

    grid_spec = pltpu.PrefetchScalarGridSpec(
        num_scalar_prefetch=1,
        grid=(G,),
        in_specs=in_specs,
        out_specs=pl.BlockSpec((1, _ARMS, 2), lambda i, sd_r: (i, 0, 0)),
    )
    out = pl.pallas_call(
        _alpha_body,
        grid_spec=grid_spec,
        out_shape=jax.ShapeDtypeStruct((G, _ARMS, 2), jnp.float32),
    )(sd, ea3, we_flat.reshape(1, hc), att2, *([xl3] * _ARMS), *([xr3] * _ARMS))
    return out.reshape(Ef, 2)


def _softmax_body(d_ref, al_ref, o_ref):
    d = d_ref[...]
    al = al_ref[...]
    n = d.shape[1]
    neg = jnp.float32(-3e38)

    def fwd(vals, dd, op, fill):
        v = vals
        s = 1
        while s < n:
            pv = jnp.concatenate(
                [jnp.full((2, s), fill, jnp.float32), v[:, :-s]], axis=1)
            pd = jnp.concatenate(
                [jnp.full((2, s), -1, jnp.int32), dd[:, :-s]], axis=1)
            v = jnp.where(dd == pd, op(v, pv), v)
            s *= 2
        return v

    def bwd(vals, dd, op, fill):
        v = vals
        s = 1
        while s < n:
            nv = jnp.concatenate(
                [v[:, s:], jnp.full((2, s), fill, jnp.float32)], axis=1)
            nd = jnp.concatenate(
                [dd[:, s:], jnp.full((2, s), -1, jnp.int32)], axis=1)
            v = jnp.where(dd == nd, op(v, nv), v)
            s *= 2
        return v

    amax = bwd(fwd(al, d, jnp.maximum, neg), d, jnp.maximum, neg)
    ex = jnp.exp(al - amax)
    den = bwd(fwd(ex, d, jnp.add, 0.0), d, jnp.maximum, neg)
    o_ref[...] = ex / (den + 1e-16)


def _seg_softmax(alpha, dst_s):
    Ef = alpha.shape[0]
    al_t = alpha.T
    d_t = jnp.broadcast_to(dst_s[None, :], (2, Ef)).astype(jnp.int32)
    out = pl.pallas_call(
        _softmax_body,
        out_shape=jax.ShapeDtypeStruct((2, Ef), jnp.float32),
    )(d_t, al_t)
    return out.T


def _gat(x, p, sd, ea_s, num_nodes, H, C):
    xl = _project(x, p["Wl"], p["bl"])
    xr = _project(x, p["Wr"], p["br"])
    we_flat = p["We"].reshape(-1)
    att_flat = p["att"].reshape(-1)

    alpha = _edge_alpha(xl, xr, sd, ea_s, we_flat, att_flat)
    a = _seg_softmax(alpha, sd[1])

    w = jnp.repeat(a, C, axis=1)
    out = jax.ops.segment_sum(xl[sd[0]] * w, sd[1], num_segments=num_nodes)
    out = out.reshape(num_nodes, H, C).mean(axis=1) + p["bias"]
    return out


def _bn(x, g, b):
    m = x.mean(axis=0)
    v = x.var(axis=0)
    return g * (x - m) / jnp.sqrt(v + 1e-5) + b


def kernel(h, edge_index, edge_weight, params):
    num_nodes = h.shape[0]
    H = params["conv1"]["att"].shape[0]
    C = params["conv1"]["att"].shape[1]
    src = edge_index[0]
    dst = edge_index[1]
    loop = jnp.arange(num_nodes, dtype=src.dtype)
    src_f = jnp.concatenate([src, loop])
    dst_f = jnp.concatenate([dst, loop])
    ea_mean = jnp.mean(edge_weight, axis=0, keepdims=True)
    ea_f = jnp.concatenate(
        [edge_weight.reshape(-1), jnp.broadcast_to(ea_mean.reshape(1), (num_nodes,))]
    )

    perm = jnp.argsort(dst_f)
    src_s = src_f[perm]
    dst_s = dst_f[perm]
    ea_s = ea_f[perm]
    sd = jnp.stack([src_s, dst_s]).astype(jnp.int32)

    x = _bn(h, params["bn0_g"], params["bn0_b"])
    x = jax.nn.relu(
        _bn(_gat(x, params["conv1"], sd, ea_s, num_nodes, H, C),
            params["bn1_g"], params["bn1_b"]))
    x = jax.nn.relu(
        _bn(_gat(x, params["conv2"], sd, ea_s, num_nodes, H, C),
            params["bn2_g"], params["bn2_b"]))
    mu = _gat(x, params["mu"], sd, ea_s, num_nodes, H, C)
    log_std = _gat(x, params["log_std"], sd, ea_s, num_nodes, H, C)
    return (mu, log_std)
